# baseline (speedup 1.0000x reference)
import functools

import jax
import jax.numpy as jnp
from jax import lax
from jax.experimental import pallas as pl
from jax.experimental.pallas import tpu as pltpu
from jax.experimental.pallas import tpu_sc as plsc

NUM_NODES = 100000
NODE_DIM = 128
MEMORY_DIM = 128
EMBED_DIM = 64
E = 500000

NC = 2
NS = 16
NW = NC * NS
CHUNK = 128
NCH = 246
NPW = CHUNK * NCH
BPAD = NW * NPW

BN = 2000
BE = 5000


def _proj_body(mem_ref, w_ref, out_ref):
    res = jnp.dot(mem_ref[...], w_ref[...], preferred_element_type=jnp.float32)
    out_ref[...] = res.astype(jnp.bfloat16)


def _edge_body(sf_ref, df_ref, gs_ref, gd_ref, w1b_ref, b1_ref, w2_ref,
               b2_ref, out_ref):
    fs = jnp.dot(sf_ref[...], w1b_ref[...], preferred_element_type=jnp.float32)
    fd = jnp.dot(df_ref[...], w1b_ref[...], preferred_element_type=jnp.float32)
    hs = jnp.maximum(fs + gs_ref[...].astype(jnp.float32) + b1_ref[...], 0.0)
    hd = jnp.maximum(fd + gd_ref[...].astype(jnp.float32) + b1_ref[...], 0.0)
    es = jnp.dot(hs, w2_ref[...], preferred_element_type=jnp.float32) + b2_ref[...]
    ed = jnp.dot(hd, w2_ref[...], preferred_element_type=jnp.float32) + b2_ref[...]
    out_ref[...] = jnp.sum(es * ed, axis=-1).reshape(1, 1, BE)


@functools.partial(
    pl.kernel,
    out_type=jax.ShapeDtypeStruct((BPAD, EMBED_DIM), jnp.bfloat16),
    mesh=plsc.VectorSubcoreMesh(core_axis_name="c", subcore_axis_name="s"),
    compiler_params=pltpu.CompilerParams(use_tc_tiling_on_sc=False),
    scratch_types=[
        pltpu.VMEM((NCH, CHUNK), jnp.int32),
        pltpu.VMEM((CHUNK, EMBED_DIM), jnp.bfloat16),
        pltpu.VMEM((CHUNK, EMBED_DIM), jnp.bfloat16),
        pltpu.SemaphoreType.DMA,
        pltpu.SemaphoreType.DMA,
    ],
)
def _sc_gather(table_hbm, idx_hbm, out_hbm, idx_v, rows_a, rows_b,
               sem_a, sem_b):
    wid = lax.axis_index("s") * NC + lax.axis_index("c")
    base = wid * NPW
    pltpu.sync_copy(idx_hbm.at[wid], idx_v)

    def body(j, carry):
        ja = 2 * j
        jb = 2 * j + 1
        ca = pltpu.make_async_copy(table_hbm.at[idx_v.at[ja]], rows_a, sem_a)
        cb = pltpu.make_async_copy(table_hbm.at[idx_v.at[jb]], rows_b, sem_b)
        ca.start()
        cb.start()
        ca.wait()
        pltpu.sync_copy(rows_a, out_hbm.at[pl.ds(base + ja * CHUNK, CHUNK)])
        cb.wait()
        pltpu.sync_copy(rows_b, out_hbm.at[pl.ds(base + jb * CHUNK, CHUNK)])
        return carry

    lax.fori_loop(0, NCH // 2, body, 0)


def kernel(src_nodes, dst_nodes, src_features, dst_features, memory,
           W1, b1, W2, b2):
    src_nodes = src_nodes.astype(jnp.int32)
    dst_nodes = dst_nodes.astype(jnp.int32)
    w1_mem = W1[:MEMORY_DIM]
    w1_feat = W1[MEMORY_DIM:]
    b1r = b1.reshape(1, EMBED_DIM)
    b2r = b2.reshape(1, EMBED_DIM)

    memory_proj = pl.pallas_call(
        _proj_body,
        grid=(NUM_NODES // BN,),
        in_specs=[
            pl.BlockSpec((BN, MEMORY_DIM), lambda i: (i, 0)),
            pl.BlockSpec((MEMORY_DIM, EMBED_DIM), lambda i: (0, 0)),
        ],
        out_specs=pl.BlockSpec((BN, EMBED_DIM), lambda i: (i, 0)),
        out_shape=jax.ShapeDtypeStruct((NUM_NODES, EMBED_DIM), jnp.bfloat16),
    )(memory, w1_mem)

    idx = jnp.concatenate(
        [src_nodes, dst_nodes,
         jnp.zeros((BPAD - 2 * E,), jnp.int32)]).reshape(NW, NCH, CHUNK)
    gathered = _sc_gather(memory_proj, idx)

    nblk = E // BE
    logits = pl.pallas_call(
        _edge_body,
        grid=(nblk,),
        in_specs=[
            pl.BlockSpec((BE, NODE_DIM), lambda i: (i, 0)),
            pl.BlockSpec((BE, NODE_DIM), lambda i: (i, 0)),
            pl.BlockSpec((BE, EMBED_DIM), lambda i: (i, 0)),
            pl.BlockSpec((BE, EMBED_DIM), lambda i: (i + nblk, 0)),
            pl.BlockSpec((NODE_DIM, EMBED_DIM), lambda i: (0, 0)),
            pl.BlockSpec((1, EMBED_DIM), lambda i: (0, 0)),
            pl.BlockSpec((EMBED_DIM, EMBED_DIM), lambda i: (0, 0)),
            pl.BlockSpec((1, EMBED_DIM), lambda i: (0, 0)),
        ],
        out_specs=pl.BlockSpec((1, 1, BE), lambda i: (i, 0, 0)),
        out_shape=jax.ShapeDtypeStruct((nblk, 1, BE), jnp.float32),
    )(src_features, dst_features, gathered, gathered, w1_feat, b1r, W2, b2r)
    return logits.reshape(E)


# speedup vs baseline: 1.5137x; 1.5137x over previous
import functools

import jax
import jax.numpy as jnp
from jax import lax
from jax.experimental import pallas as pl
from jax.experimental.pallas import tpu as pltpu
from jax.experimental.pallas import tpu_sc as plsc

NUM_NODES = 100000
NODE_DIM = 128
MEMORY_DIM = 128
EMBED_DIM = 64
E = 500000

NC = 2
NS = 16
NW = NC * NS
CHUNK = 128
NCH = 246
NPW = CHUNK * NCH
BPAD = NW * NPW

BN = 2000
BE = 4000
HB = BE // 2


def _proj_body(mem_ref, w_ref, out_ref):
    out_ref[...] = jnp.dot(mem_ref[...], w_ref[...],
                           preferred_element_type=jnp.float32)


def _edge_body(sf_ref, df_ref, gs_ref, gd_ref, w1b_ref, b1_ref, w2_ref,
               b2_ref, out_ref):
    fs = jnp.dot(sf_ref[...], w1b_ref[...], preferred_element_type=jnp.float32)
    fd = jnp.dot(df_ref[...], w1b_ref[...], preferred_element_type=jnp.float32)
    g2s = gs_ref[...]
    g2d = gd_ref[...]
    b1v = b1_ref[...]
    w2 = w2_ref[...]
    b2v = b2_ref[...]

    def emb(h):
        return jnp.dot(h, w2, preferred_element_type=jnp.float32) + b2v

    hs_lo = jnp.maximum(g2s[:, :EMBED_DIM] + fs[:HB] + b1v, 0.0)
    hs_hi = jnp.maximum(g2s[:, EMBED_DIM:] + fs[HB:] + b1v, 0.0)
    hd_lo = jnp.maximum(g2d[:, :EMBED_DIM] + fd[:HB] + b1v, 0.0)
    hd_hi = jnp.maximum(g2d[:, EMBED_DIM:] + fd[HB:] + b1v, 0.0)
    l_lo = jnp.sum(emb(hs_lo) * emb(hd_lo), axis=-1)
    l_hi = jnp.sum(emb(hs_hi) * emb(hd_hi), axis=-1)
    out_ref[...] = jnp.concatenate([l_lo, l_hi]).reshape(1, 1, BE)


@functools.partial(
    pl.kernel,
    out_type=jax.ShapeDtypeStruct((BPAD, EMBED_DIM), jnp.float32),
    mesh=plsc.VectorSubcoreMesh(core_axis_name="c", subcore_axis_name="s"),
    compiler_params=pltpu.CompilerParams(use_tc_tiling_on_sc=False),
    scratch_types=[
        pltpu.VMEM((NCH, CHUNK), jnp.int32),
        pltpu.VMEM((CHUNK, EMBED_DIM), jnp.float32),
        pltpu.VMEM((CHUNK, EMBED_DIM), jnp.float32),
        pltpu.SemaphoreType.DMA,
        pltpu.SemaphoreType.DMA,
    ],
)
def _sc_gather(table_hbm, idx_hbm, out_hbm, idx_v, rows_a, rows_b,
               sem_a, sem_b):
    wid = lax.axis_index("s") * NC + lax.axis_index("c")
    base = wid * NPW
    pltpu.sync_copy(idx_hbm.at[wid], idx_v)

    def body(j, carry):
        ja = 2 * j
        jb = 2 * j + 1
        ca = pltpu.make_async_copy(table_hbm.at[idx_v.at[ja]], rows_a, sem_a)
        cb = pltpu.make_async_copy(table_hbm.at[idx_v.at[jb]], rows_b, sem_b)
        ca.start()
        cb.start()
        ca.wait()
        pltpu.sync_copy(rows_a, out_hbm.at[pl.ds(base + ja * CHUNK, CHUNK)])
        cb.wait()
        pltpu.sync_copy(rows_b, out_hbm.at[pl.ds(base + jb * CHUNK, CHUNK)])
        return carry

    lax.fori_loop(0, NCH // 2, body, 0)


def _pair_permute(idx):
    return idx.reshape(E // BE, 2, HB).transpose(0, 2, 1).reshape(E)


def kernel(src_nodes, dst_nodes, src_features, dst_features, memory,
           W1, b1, W2, b2):
    src_nodes = src_nodes.astype(jnp.int32)
    dst_nodes = dst_nodes.astype(jnp.int32)
    w1_mem = W1[:MEMORY_DIM]
    w1_feat = W1[MEMORY_DIM:]
    b1r = b1.reshape(1, EMBED_DIM)
    b2r = b2.reshape(1, EMBED_DIM)

    memory_proj = pl.pallas_call(
        _proj_body,
        grid=(NUM_NODES // BN,),
        in_specs=[
            pl.BlockSpec((BN, MEMORY_DIM), lambda i: (i, 0)),
            pl.BlockSpec((MEMORY_DIM, EMBED_DIM), lambda i: (0, 0)),
        ],
        out_specs=pl.BlockSpec((BN, EMBED_DIM), lambda i: (i, 0)),
        out_shape=jax.ShapeDtypeStruct((NUM_NODES, EMBED_DIM), jnp.float32),
    )(memory, w1_mem)

    idx = jnp.concatenate(
        [_pair_permute(src_nodes), _pair_permute(dst_nodes),
         jnp.zeros((BPAD - 2 * E,), jnp.int32)]).reshape(NW, NCH, CHUNK)
    gathered = _sc_gather(memory_proj, idx)
    g2 = gathered.reshape(BPAD // 2, 2 * EMBED_DIM)

    nblk = E // BE
    logits = pl.pallas_call(
        _edge_body,
        grid=(nblk,),
        in_specs=[
            pl.BlockSpec((BE, NODE_DIM), lambda i: (i, 0)),
            pl.BlockSpec((BE, NODE_DIM), lambda i: (i, 0)),
            pl.BlockSpec((HB, 2 * EMBED_DIM), lambda i: (i, 0)),
            pl.BlockSpec((HB, 2 * EMBED_DIM), lambda i: (i + nblk, 0)),
            pl.BlockSpec((NODE_DIM, EMBED_DIM), lambda i: (0, 0)),
            pl.BlockSpec((1, EMBED_DIM), lambda i: (0, 0)),
            pl.BlockSpec((EMBED_DIM, EMBED_DIM), lambda i: (0, 0)),
            pl.BlockSpec((1, EMBED_DIM), lambda i: (0, 0)),
        ],
        out_specs=pl.BlockSpec((1, 1, BE), lambda i: (i, 0, 0)),
        out_shape=jax.ShapeDtypeStruct((nblk, 1, BE), jnp.float32),
    )(src_features, dst_features, g2, g2, w1_feat, b1r, W2, b2r)
    return logits.reshape(E)
